# baseline (speedup 1.0000x reference)
import jax
import jax.numpy as jnp
from jax import lax
from jax.experimental import pallas as pl
from jax.experimental.pallas import tpu as pltpu
from jax.experimental.pallas import tpu_sc as plsc

B = 16384
HL = 50
D = 32
DB = 16
NC, NS = 2, 16
NW = NC * NS
BPW = B // NW
CH = 128
NCH = BPW // CH
NF = 6
HCH = HL * NCH
PR = BPW // 4
PRB = BPW // 8

_mesh = plsc.VectorSubcoreMesh(core_axis_name="c", subcore_axis_name="s")


def _body(num_h, idx_h, histT_h, wb_h,
          Ec0, Ec1, Ec2, Ec3, Eb0, Eb1, Eh,
          o_num, o_c0, o_c1, o_c2, o_c3, o_b0, o_b1, o_h,
          idx_v, histT_v, num_v, wb_v, stgA, stgB,
          gA, gB, fbuf, bb0, bb1, acc, inv_v,
          sin, sgA, sgB, sout):
    cid = lax.axis_index("c")
    sid = lax.axis_index("s")
    wid = sid * NC + cid

    gbuf = (gA, gB)
    stg = (stgA, stgB)
    sg = (sgA, sgB)
    m3 = jnp.full((16,), 3, jnp.int32)
    m7 = jnp.full((16,), 7, jnp.int32)

    ins = [
        pltpu.async_copy(idx_h.at[pl.ds(wid * NF * NCH, NF * NCH)], idx_v, sin),
        pltpu.async_copy(histT_h.at[pl.ds(wid * HCH, HCH)], histT_v, sin),
        pltpu.async_copy(num_h.at[pl.ds(wid * PRB, PRB)], num_v, sin),
        pltpu.async_copy(wb_h, wb_v, sin),
    ]
    for cp in ins:
        cp.wait()

    def stage_rows(src_ref, row, rsh, dst_ref):
        rshv = jnp.full((16,), rsh, jnp.int32)
        three = jnp.full((16,), 3, jnp.int32)
        seven = jnp.full((16,), 7, jnp.int32)

        def sbody(v, carry):
            iv = src_ref[row, pl.ds(v * 16, 16)]
            dst_ref[pl.ds(v * 16, 16)] = lax.bitwise_or(
                lax.shift_left(lax.shift_right_logical(iv, rshv), three),
                lax.bitwise_and(iv, seven))
            return carry
        lax.fori_loop(0, 8, sbody, 0)

    def fire(table, p):
        return pltpu.async_copy(table.at[stg[p]], gbuf[p], sg[p])

    stage_rows(idx_v, 0, 5, stgA)
    dA = fire(Ec0, 0)
    stage_rows(idx_v, 1, 5, stgB)
    dB = fire(Ec0, 1)

    wvec = [(wb_v[k // 4, pl.ds((k % 4) * 32, 16)],
             wb_v[k // 4, pl.ds((k % 4) * 32 + 16, 16)]) for k in range(13)]
    bv0 = wb_v[3, pl.ds(32, 16)]
    bv1 = wb_v[3, pl.ds(48, 16)]

    def num_body(rr, carry):
        for s in range(4):
            r = rr * 4 + s
            nv = num_v[lax.div(r, 8), pl.ds(lax.rem(r, 8) * 16, 16)]
            a0, a1 = bv0, bv1
            for k in range(13):
                sv = jnp.broadcast_to(nv[k], (16,))
                a0 = a0 + sv * wvec[k][0]
                a1 = a1 + sv * wvec[k][1]
            fbuf[rr, pl.ds(s * 32, 16)] = a0
            fbuf[rr, pl.ds(s * 32 + 16, 16)] = a1
        return carry

    lax.fori_loop(0, PR, num_body, 0)
    wprev = pltpu.async_copy(fbuf, o_num.at[pl.ds(wid * PR, PR)], sout)

    def extract_chunk(p, f, q, s3, outbuf):
        mask = m7 if s3 else m3
        w = DB if s3 else D
        def ebody(g, carry):
            iq = lax.bitwise_and(
                lax.shift_right_logical(idx_v[f * NCH + q, pl.ds(g * 16, 16)],
                                        jnp.full((16,), 3, jnp.int32)), mask)
            for i in range(16):
                col = pl.multiple_of(iq[i] * w, 16)
                if s3:
                    r = q * (CH // 8) + g * 2 + i // 8
                    oc = (i % 8) * 16
                    outbuf[r, pl.ds(oc, 16)] = gbuf[p][g * 16 + i,
                                                       pl.ds(col, 16)]
                else:
                    r = q * (CH // 4) + g * 4 + i // 4
                    oc = (i % 4) * 32
                    outbuf[r, pl.ds(oc, 16)] = gbuf[p][g * 16 + i,
                                                       pl.ds(col, 16)]
                    outbuf[r, pl.ds(oc + 16, 16)] = gbuf[p][g * 16 + i,
                                                            pl.ds(col + 16, 16)]
            return carry
        lax.fori_loop(0, 8, ebody, 0)

    fields = [
        (Ec0, 5, False, fbuf, o_c0),
        (Ec1, 5, False, fbuf, o_c1),
        (Ec2, 5, False, fbuf, o_c2),
        (Ec3, 5, False, fbuf, o_c3),
        (Eb0, 6, True, bb0, o_b0),
        (Eb1, 6, True, bb1, o_b1),
    ]

    pend = [dA, dB]
    NT = NF * NCH
    for t in range(NT):
        f, q = t // NCH, t % NCH
        table, shift, s3, outbuf, outarr = fields[f]
        p = t % 2
        pend[p].wait()
        if q == 0:
            wprev.wait()
        extract_chunk(p, f, q, s3, outbuf)
        if t + 2 < NT:
            nf, nq = (t + 2) // NCH, (t + 2) % NCH
            ntable, nshift = fields[nf][0], fields[nf][1]
            stage_rows(idx_v, nf * NCH + nq, nshift, stg[p])
            pend[p] = fire(ntable, p)
        if q == NCH - 1:
            if s3:
                wprev = pltpu.async_copy(
                    outbuf, outarr.at[pl.ds(wid * PRB, PRB)], sout)
            else:
                wprev = pltpu.async_copy(
                    outbuf, outarr.at[pl.ds(wid * PR, PR)], sout)

    stage_rows(histT_v, 0, 5, stgA)
    fire(Eh, 0)
    stage_rows(histT_v, 1, 5, stgB)
    fire(Eh, 1)

    def zero_body(r, carry):
        z = jnp.zeros((16,), jnp.float32)
        for v in range(8):
            acc[r, pl.ds(v * 16, 16)] = z
        return carry

    lax.fori_loop(0, PR, zero_body, 0)

    def hist_step(k, carry):
        for p in range(2):
            c = 2 * k + p
            pltpu.make_async_copy(Eh.at[pl.ds(0, CH)], gbuf[p], sg[p]).wait()
            roff4 = pl.multiple_of(lax.rem(c, NCH) * (CH // 4), CH // 4)

            def abody(g, carry2):
                iq = lax.bitwise_and(
                    lax.shift_right_logical(histT_v[c, pl.ds(g * 16, 16)],
                                            jnp.full((16,), 3, jnp.int32)), m3)
                for i in range(16):
                    col = pl.multiple_of(iq[i] * D, 16)
                    rr = roff4 + g * 4 + i // 4
                    oc = (i % 4) * 32
                    a0 = gbuf[p][g * 16 + i, pl.ds(col, 16)]
                    a1 = gbuf[p][g * 16 + i, pl.ds(col + 16, 16)]
                    plsc.addupdate(acc.at[rr, pl.ds(oc, 16)], a0)
                    plsc.addupdate(acc.at[rr, pl.ds(oc + 16, 16)], a1)
                return carry2

            lax.fori_loop(0, 8, abody, 0)

            @pl.when(c + 2 < HCH)
            def _():
                stage_rows(histT_v, c + 2, 5, stg[p])
                fire(Eh, p)
        return carry

    lax.fori_loop(0, HCH // 2, hist_step, 0)

    def cnt_body(g, carry):
        q = lax.div(g, CH // 16)
        off = lax.rem(g, CH // 16) * 16
        c = jnp.zeros((16,), jnp.float32)
        zi = jnp.zeros((16,), jnp.int32)
        one = jnp.full((16,), 1.0, jnp.float32)
        zf = jnp.zeros((16,), jnp.float32)
        for j in range(HL):
            iv = histT_v[j * NCH + q, pl.ds(off, 16)]
            c = c + jnp.where(iv != zi, one, zf)
        inv_v[pl.ds(g * 16, 16)] = one / jnp.maximum(
            c, jnp.full((16,), 1e-6, jnp.float32))
        return carry

    lax.fori_loop(0, BPW // 16, cnt_body, 0)

    def scale_body(g, carry):
        iv = inv_v[pl.ds(g * 16, 16)]
        for i in range(16):
            sv = jnp.broadcast_to(iv[i], (16,))
            rr = g * 4 + i // 4
            oc = (i % 4) * 32
            acc[rr, pl.ds(oc, 16)] = acc[rr, pl.ds(oc, 16)] * sv
            acc[rr, pl.ds(oc + 16, 16)] = acc[rr, pl.ds(oc + 16, 16)] * sv
        return carry

    lax.fori_loop(0, BPW // 16, scale_body, 0)
    wh = pltpu.async_copy(acc, o_h.at[pl.ds(wid * PR, PR)], sout)
    wprev.wait()
    wh.wait()


_WIDE = jax.ShapeDtypeStruct((B // 4, 128), jnp.float32)
_NARROW = jax.ShapeDtypeStruct((B // 8, 128), jnp.float32)

_encode = pl.kernel(
    _body,
    out_type=[_WIDE, _WIDE, _WIDE, _WIDE, _WIDE, _NARROW, _NARROW, _WIDE],
    mesh=_mesh,
    scratch_types=[
        pltpu.VMEM((NF * NCH, CH), jnp.int32),
        pltpu.VMEM((HCH, CH), jnp.int32),
        pltpu.VMEM((PRB, 128), jnp.float32),
        pltpu.VMEM((4, 128), jnp.float32),
        pltpu.VMEM((CH,), jnp.int32),
        pltpu.VMEM((CH,), jnp.int32),
        pltpu.VMEM((CH, CH), jnp.float32),
        pltpu.VMEM((CH, CH), jnp.float32),
        pltpu.VMEM((PR, 128), jnp.float32),
        pltpu.VMEM((PRB, 128), jnp.float32),
        pltpu.VMEM((PRB, 128), jnp.float32),
        pltpu.VMEM((PR, 128), jnp.float32),
        pltpu.VMEM((BPW,), jnp.float32),
        pltpu.SemaphoreType.DMA,
        pltpu.SemaphoreType.DMA,
        pltpu.SemaphoreType.DMA,
        pltpu.SemaphoreType.DMA,
    ],
)


def kernel(numeric, cat_0, cat_1, cat_2, cat_3, bkt_0, bkt_1, hist_items,
           W_num, b_num, E_cat_0, E_cat_1, E_cat_2, E_cat_3,
           E_bkt_0, E_bkt_1, E_hist):
    num_p = jnp.pad(numeric, ((0, 0), (0, 3))).reshape(B * 16 // 128, 128)
    idx_all = jnp.stack([cat_0, cat_1, cat_2, cat_3, bkt_0, bkt_1])
    idx_all = idx_all.reshape(NF, NW, BPW).transpose(1, 0, 2)
    idx_all = idx_all.reshape(NW * NF * NCH, CH)
    hist_t = jnp.transpose(hist_items).reshape(HL, NW, BPW).transpose(1, 0, 2)
    hist_t = hist_t.reshape(NW * HCH, CH)
    wb = jnp.concatenate([jnp.transpose(W_num), b_num[None, :],
                          jnp.zeros((2, 32), jnp.float32)], axis=0)
    wb = wb.reshape(4, 128)
    def wide(t):
        n = t.shape[0]
        t4 = t.reshape(n // 32, 4, 8, 32).transpose(0, 2, 1, 3)
        return t4.reshape(n // 4, 128)

    def wideb(t):
        t = jnp.pad(t, ((0, 1024 - t.shape[0]), (0, 0)))
        t8 = t.reshape(16, 8, 8, 16).transpose(0, 2, 1, 3)
        return t8.reshape(128, 128)

    outs = _encode(num_p, idx_all, hist_t, wb,
                   wide(E_cat_0), wide(E_cat_1), wide(E_cat_2), wide(E_cat_3),
                   wideb(E_bkt_0), wideb(E_bkt_1), wide(E_hist))
    widths = (D, D, D, D, D, DB, DB, D)
    flat = [o.reshape(B, w) for o, w in zip(outs, widths)]
    return jnp.concatenate(flat, axis=-1)


# speedup vs baseline: 1.5622x; 1.5622x over previous
import functools

import jax
import jax.numpy as jnp
from jax import lax
from jax.experimental import pallas as pl
from jax.experimental.pallas import tpu as pltpu
from jax.experimental.pallas import tpu_sc as plsc

B = 16384
HL = 50
D = 32
DB = 16
OUT_D = 224
NC, NS = 2, 16
NW = NC * NS
BPW = B // NW
CH = 128
NCH = BPW // CH

COL_NUM, COL_C0, COL_C1, COL_C2, COL_C3 = 0, 32, 64, 96, 128
COL_B0, COL_B1, COL_H = 160, 176, 192

_mesh = plsc.VectorSubcoreMesh(core_axis_name="c", subcore_axis_name="s")


def _body(num_h, c0_h, c1_h, c2_h, c3_h, b0_h, b1_h, histT_h, wt_h, b_h,
          Ec0, Ec1, Ec2, Ec3, Eb0, Eb1, Eh,
          o_num, o_c0, o_c1, o_c2, o_c3, o_b0, o_b1, o_h,
          idx0, idx1, idx2, idx3, ib0, ib1, histT_v, num_v, wt_v, b_v,
          bufA, bufB, bb0, bb1, acc, num_out, inv_v,
          sin, s0, s1, s2, sout, sh0, sh1):
    cid = lax.axis_index("c")
    sid = lax.axis_index("s")
    wid = sid * NC + cid
    base = wid * BPW
    qbase = wid * NCH

    ins = [
        pltpu.async_copy(c0_h.at[pl.ds(qbase, NCH)], idx0, sin),
        pltpu.async_copy(c1_h.at[pl.ds(qbase, NCH)], idx1, sin),
        pltpu.async_copy(c2_h.at[pl.ds(qbase, NCH)], idx2, sin),
        pltpu.async_copy(c3_h.at[pl.ds(qbase, NCH)], idx3, sin),
        pltpu.async_copy(b0_h.at[pl.ds(qbase, NCH)], ib0, sin),
        pltpu.async_copy(b1_h.at[pl.ds(qbase, NCH)], ib1, sin),
        pltpu.async_copy(histT_h.at[:, pl.ds(qbase, NCH), :], histT_v, sin),
        pltpu.async_copy(num_h.at[pl.ds(base, BPW)], num_v, sin),
        pltpu.async_copy(wt_h, wt_v, sin),
        pltpu.async_copy(b_h, b_v, sin),
    ]
    for cp in ins:
        cp.wait()

    def gather(table, idx, dst, sem):
        ds = [pltpu.async_copy(table.at[idx.at[q]],
                               dst.at[pl.ds(q * CH, CH), :], sem)
              for q in range(NCH)]
        return ds

    def wait_all(ds):
        for d in ds:
            d.wait()

    g0 = gather(Ec0, idx0, bufA, s0)
    g1 = gather(Ec1, idx1, bufB, s1)
    gb0 = gather(Eb0, ib0, bb0, s2)
    gb1 = gather(Eb1, ib1, bb1, s2)
    wait_all(g0)
    w0 = pltpu.async_copy(bufA, o_c0.at[pl.ds(base, BPW)], sout)
    wait_all(g1)
    w1 = pltpu.async_copy(bufB, o_c1.at[pl.ds(base, BPW)], sout)
    w0.wait()
    g2 = gather(Ec2, idx2, bufA, s0)
    w1.wait()
    g3 = gather(Ec3, idx3, bufB, s1)
    wait_all(gb0)
    wb0 = pltpu.async_copy(bb0, o_b0.at[pl.ds(base, BPW)], sout)
    wait_all(gb1)
    wb1 = pltpu.async_copy(bb1, o_b1.at[pl.ds(base, BPW)], sout)
    wait_all(g2)
    w2 = pltpu.async_copy(bufA, o_c2.at[pl.ds(base, BPW)], sout)
    wait_all(g3)
    w3 = pltpu.async_copy(bufB, o_c3.at[pl.ds(base, BPW)], sout)

    wvec = [(wt_v[k, pl.ds(0, 16)], wt_v[k, pl.ds(16, 16)]) for k in range(13)]
    bv0 = b_v[pl.ds(0, 16)]
    bv1 = b_v[pl.ds(16, 16)]

    def num_body(r, carry):
        nv = num_v[r, pl.ds(0, 16)]
        a0, a1 = bv0, bv1
        for k in range(13):
            sv = jnp.broadcast_to(nv[k], (16,))
            a0 = a0 + sv * wvec[k][0]
            a1 = a1 + sv * wvec[k][1]
        num_out[r, pl.ds(0, 16)] = a0
        num_out[r, pl.ds(16, 16)] = a1
        return carry

    lax.fori_loop(0, BPW, num_body, 0, unroll=2)
    wn = pltpu.async_copy(num_out, o_num.at[pl.ds(base, BPW)], sout)

    w2.wait()
    w3.wait()
    gbuf = (bufA, bufB)
    shs = (sh0, sh1)

    def hist_gather(j, p):
        return [pltpu.async_copy(Eh.at[histT_v.at[j, q]],
                                 gbuf[p].at[pl.ds(q * CH, CH), :], shs[p])
                for q in range(NCH)]

    def mk_acc_loop(gb, first):
        def acc_body(r, carry):
            g0v = gb[r, pl.ds(0, 16)]
            g1v = gb[r, pl.ds(16, 16)]
            if first:
                acc[r, pl.ds(0, 16)] = g0v
                acc[r, pl.ds(16, 16)] = g1v
            else:
                plsc.addupdate(acc.at[r, pl.ds(0, 16)], g0v)
                plsc.addupdate(acc.at[r, pl.ds(16, 16)], g1v)
            return carry
        return acc_body

    pend = [hist_gather(0, 0), hist_gather(1, 1)]
    for j in range(HL):
        p = j % 2
        wait_all(pend[p])
        lax.fori_loop(0, BPW, mk_acc_loop(gbuf[p], j == 0), 0, unroll=8)
        if j + 2 < HL:
            pend[p] = hist_gather(j + 2, p)

    def cnt_body(g, carry):
        q = g // (CH // 16)
        off = (g % (CH // 16)) * 16
        c = jnp.zeros((16,), jnp.float32)
        zi = jnp.zeros((16,), jnp.int32)
        one = jnp.full((16,), 1.0, jnp.float32)
        zf = jnp.zeros((16,), jnp.float32)
        for j in range(HL):
            iv = histT_v[j, q, pl.ds(off, 16)]
            c = c + jnp.where(iv != zi, one, zf)
        inv_v[pl.ds(g * 16, 16)] = one / jnp.maximum(c, jnp.full((16,), 1e-6, jnp.float32))
        return carry

    lax.fori_loop(0, BPW // 16, cnt_body, 0)

    def scale_body(g, carry):
        iv = inv_v[pl.ds(g * 16, 16)]
        for i in range(16):
            sv = jnp.broadcast_to(iv[i], (16,))
            r = g * 16 + i
            acc[r, pl.ds(0, 16)] = acc[r, pl.ds(0, 16)] * sv
            acc[r, pl.ds(16, 16)] = acc[r, pl.ds(16, 16)] * sv
        return carry

    lax.fori_loop(0, BPW // 16, scale_body, 0)
    wh = pltpu.async_copy(acc, o_h.at[pl.ds(base, BPW)], sout)

    for d in (wb0, wb1, wn, wh):
        d.wait()


_encode = pl.kernel(
    _body,
    out_type=[
        jax.ShapeDtypeStruct((B, D), jnp.float32),
        jax.ShapeDtypeStruct((B, D), jnp.float32),
        jax.ShapeDtypeStruct((B, D), jnp.float32),
        jax.ShapeDtypeStruct((B, D), jnp.float32),
        jax.ShapeDtypeStruct((B, D), jnp.float32),
        jax.ShapeDtypeStruct((B, DB), jnp.float32),
        jax.ShapeDtypeStruct((B, DB), jnp.float32),
        jax.ShapeDtypeStruct((B, D), jnp.float32),
    ],
    mesh=_mesh,
    compiler_params=pltpu.CompilerParams(use_tc_tiling_on_sc=False),
    scratch_types=[
        pltpu.VMEM((NCH, CH), jnp.int32),
        pltpu.VMEM((NCH, CH), jnp.int32),
        pltpu.VMEM((NCH, CH), jnp.int32),
        pltpu.VMEM((NCH, CH), jnp.int32),
        pltpu.VMEM((NCH, CH), jnp.int32),
        pltpu.VMEM((NCH, CH), jnp.int32),
        pltpu.VMEM((HL, NCH, CH), jnp.int32),
        pltpu.VMEM((BPW, 16), jnp.float32),
        pltpu.VMEM((13, D), jnp.float32),
        pltpu.VMEM((D,), jnp.float32),
        pltpu.VMEM((BPW, D), jnp.float32),
        pltpu.VMEM((BPW, D), jnp.float32),
        pltpu.VMEM((BPW, DB), jnp.float32),
        pltpu.VMEM((BPW, DB), jnp.float32),
        pltpu.VMEM((BPW, D), jnp.float32),
        pltpu.VMEM((BPW, D), jnp.float32),
        pltpu.VMEM((BPW,), jnp.float32),
        pltpu.SemaphoreType.DMA,
        pltpu.SemaphoreType.DMA,
        pltpu.SemaphoreType.DMA,
        pltpu.SemaphoreType.DMA,
        pltpu.SemaphoreType.DMA,
        pltpu.SemaphoreType.DMA,
        pltpu.SemaphoreType.DMA,
    ],
)


def kernel(numeric, cat_0, cat_1, cat_2, cat_3, bkt_0, bkt_1, hist_items,
           W_num, b_num, E_cat_0, E_cat_1, E_cat_2, E_cat_3,
           E_bkt_0, E_bkt_1, E_hist):
    numeric = jnp.pad(numeric, ((0, 0), (0, 3)))
    hist_T = jnp.transpose(hist_items).reshape(HL, B // CH, CH)
    c0 = cat_0.reshape(B // CH, CH)
    c1 = cat_1.reshape(B // CH, CH)
    c2 = cat_2.reshape(B // CH, CH)
    c3 = cat_3.reshape(B // CH, CH)
    b0 = bkt_0.reshape(B // CH, CH)
    b1 = bkt_1.reshape(B // CH, CH)
    w_t = jnp.transpose(W_num)
    outs = _encode(numeric, c0, c1, c2, c3, b0, b1, hist_T, w_t, b_num,
                   E_cat_0, E_cat_1, E_cat_2, E_cat_3, E_bkt_0, E_bkt_1,
                   E_hist)
    return jnp.concatenate(outs, axis=-1)


# speedup vs baseline: 1.5672x; 1.0032x over previous
import functools

import jax
import jax.numpy as jnp
from jax import lax
from jax.experimental import pallas as pl
from jax.experimental.pallas import tpu as pltpu
from jax.experimental.pallas import tpu_sc as plsc

B = 16384
HL = 50
D = 32
DB = 16
OUT_D = 224
NC, NS = 2, 16
NW = NC * NS
BPW = B // NW
CH = 128
NCH = BPW // CH

COL_NUM, COL_C0, COL_C1, COL_C2, COL_C3 = 0, 32, 64, 96, 128
COL_B0, COL_B1, COL_H = 160, 176, 192

_mesh = plsc.VectorSubcoreMesh(core_axis_name="c", subcore_axis_name="s")


def _body(num_h, c0_h, c1_h, c2_h, c3_h, b0_h, b1_h, histT_h, wt_h, b_h,
          Ec0, Ec1, Ec2, Ec3, Eb0, Eb1, Eh,
          o_num, o_c0, o_c1, o_c2, o_c3, o_b0, o_b1, o_h,
          idx0, idx1, idx2, idx3, ib0, ib1, histT_v, num_v, wt_v, b_v,
          bufA, bufB, bb0, bb1, acc, num_out, inv_v,
          sin, s0, s1, s2, sout, sh0, sh1):
    cid = lax.axis_index("c")
    sid = lax.axis_index("s")
    wid = sid * NC + cid
    base = wid * BPW
    qbase = wid * NCH

    ins = [
        pltpu.async_copy(c0_h.at[pl.ds(qbase, NCH)], idx0, sin),
        pltpu.async_copy(c1_h.at[pl.ds(qbase, NCH)], idx1, sin),
        pltpu.async_copy(c2_h.at[pl.ds(qbase, NCH)], idx2, sin),
        pltpu.async_copy(c3_h.at[pl.ds(qbase, NCH)], idx3, sin),
        pltpu.async_copy(b0_h.at[pl.ds(qbase, NCH)], ib0, sin),
        pltpu.async_copy(b1_h.at[pl.ds(qbase, NCH)], ib1, sin),
        pltpu.async_copy(histT_h.at[:, pl.ds(qbase, NCH), :], histT_v, sin),
        pltpu.async_copy(num_h.at[pl.ds(base, BPW)], num_v, sin),
        pltpu.async_copy(wt_h, wt_v, sin),
        pltpu.async_copy(b_h, b_v, sin),
    ]
    for cp in ins:
        cp.wait()

    def gather(table, idx, dst, sem):
        ds = [pltpu.async_copy(table.at[idx.at[q]],
                               dst.at[pl.ds(q * CH, CH), :], sem)
              for q in range(NCH)]
        return ds

    def wait_all(ds):
        for d in ds:
            d.wait()

    g0 = gather(Ec0, idx0, bufA, s0)
    g1 = gather(Ec1, idx1, bufB, s1)
    gb0 = gather(Eb0, ib0, bb0, s2)
    gb1 = gather(Eb1, ib1, bb1, s2)
    wait_all(g0)
    w0 = pltpu.async_copy(bufA, o_c0.at[pl.ds(base, BPW)], sout)
    wait_all(g1)
    w1 = pltpu.async_copy(bufB, o_c1.at[pl.ds(base, BPW)], sout)
    w0.wait()
    g2 = gather(Ec2, idx2, bufA, s0)
    w1.wait()
    g3 = gather(Ec3, idx3, bufB, s1)
    wait_all(gb0)
    wb0 = pltpu.async_copy(bb0, o_b0.at[pl.ds(base, BPW)], sout)
    wait_all(gb1)
    wb1 = pltpu.async_copy(bb1, o_b1.at[pl.ds(base, BPW)], sout)
    wait_all(g2)
    w2 = pltpu.async_copy(bufA, o_c2.at[pl.ds(base, BPW)], sout)
    wait_all(g3)
    w3 = pltpu.async_copy(bufB, o_c3.at[pl.ds(base, BPW)], sout)

    wvec = [(wt_v[k, pl.ds(0, 16)], wt_v[k, pl.ds(16, 16)]) for k in range(13)]
    bv0 = b_v[pl.ds(0, 16)]
    bv1 = b_v[pl.ds(16, 16)]

    def num_body(r, carry):
        nv = num_v[r, pl.ds(0, 16)]
        a0, a1 = bv0, bv1
        for k in range(13):
            sv = jnp.broadcast_to(nv[k], (16,))
            a0 = a0 + sv * wvec[k][0]
            a1 = a1 + sv * wvec[k][1]
        num_out[r, pl.ds(0, 16)] = a0
        num_out[r, pl.ds(16, 16)] = a1
        return carry

    lax.fori_loop(0, BPW, num_body, 0, unroll=2)
    wn = pltpu.async_copy(num_out, o_num.at[pl.ds(base, BPW)], sout)

    w2.wait()
    w3.wait()
    wn.wait()
    gbuf = (bufA, bufB, num_out)
    shs = (sh0, sh1, s0)

    def hist_gather(j, p):
        return [pltpu.async_copy(Eh.at[histT_v.at[j, q]],
                                 gbuf[p].at[pl.ds(q * CH, CH), :], shs[p])
                for q in range(NCH)]

    def mk_acc_loop(gb, first):
        def acc_body(r, carry):
            g0v = gb[r, pl.ds(0, 16)]
            g1v = gb[r, pl.ds(16, 16)]
            if first:
                acc[r, pl.ds(0, 16)] = g0v
                acc[r, pl.ds(16, 16)] = g1v
            else:
                plsc.addupdate(acc.at[r, pl.ds(0, 16)], g0v)
                plsc.addupdate(acc.at[r, pl.ds(16, 16)], g1v)
            return carry
        return acc_body

    pend = [hist_gather(0, 0), hist_gather(1, 1), hist_gather(2, 2)]
    for j in range(HL):
        p = j % 3
        wait_all(pend[p])
        lax.fori_loop(0, BPW, mk_acc_loop(gbuf[p], j == 0), 0, unroll=8)
        if j + 3 < HL:
            pend[p] = hist_gather(j + 3, p)

    def cnt_body(g, carry):
        q = g // (CH // 16)
        off = (g % (CH // 16)) * 16
        c = jnp.zeros((16,), jnp.float32)
        zi = jnp.zeros((16,), jnp.int32)
        one = jnp.full((16,), 1.0, jnp.float32)
        zf = jnp.zeros((16,), jnp.float32)
        for j in range(HL):
            iv = histT_v[j, q, pl.ds(off, 16)]
            c = c + jnp.where(iv != zi, one, zf)
        inv_v[pl.ds(g * 16, 16)] = one / jnp.maximum(c, jnp.full((16,), 1e-6, jnp.float32))
        return carry

    lax.fori_loop(0, BPW // 16, cnt_body, 0)

    def scale_body(g, carry):
        iv = inv_v[pl.ds(g * 16, 16)]
        for i in range(16):
            sv = jnp.broadcast_to(iv[i], (16,))
            r = g * 16 + i
            acc[r, pl.ds(0, 16)] = acc[r, pl.ds(0, 16)] * sv
            acc[r, pl.ds(16, 16)] = acc[r, pl.ds(16, 16)] * sv
        return carry

    lax.fori_loop(0, BPW // 16, scale_body, 0)
    wh = pltpu.async_copy(acc, o_h.at[pl.ds(base, BPW)], sout)

    for d in (wb0, wb1, wh):
        d.wait()


_encode = pl.kernel(
    _body,
    out_type=[
        jax.ShapeDtypeStruct((B, D), jnp.float32),
        jax.ShapeDtypeStruct((B, D), jnp.float32),
        jax.ShapeDtypeStruct((B, D), jnp.float32),
        jax.ShapeDtypeStruct((B, D), jnp.float32),
        jax.ShapeDtypeStruct((B, D), jnp.float32),
        jax.ShapeDtypeStruct((B, DB), jnp.float32),
        jax.ShapeDtypeStruct((B, DB), jnp.float32),
        jax.ShapeDtypeStruct((B, D), jnp.float32),
    ],
    mesh=_mesh,
    compiler_params=pltpu.CompilerParams(use_tc_tiling_on_sc=False),
    scratch_types=[
        pltpu.VMEM((NCH, CH), jnp.int32),
        pltpu.VMEM((NCH, CH), jnp.int32),
        pltpu.VMEM((NCH, CH), jnp.int32),
        pltpu.VMEM((NCH, CH), jnp.int32),
        pltpu.VMEM((NCH, CH), jnp.int32),
        pltpu.VMEM((NCH, CH), jnp.int32),
        pltpu.VMEM((HL, NCH, CH), jnp.int32),
        pltpu.VMEM((BPW, 16), jnp.float32),
        pltpu.VMEM((13, D), jnp.float32),
        pltpu.VMEM((D,), jnp.float32),
        pltpu.VMEM((BPW, D), jnp.float32),
        pltpu.VMEM((BPW, D), jnp.float32),
        pltpu.VMEM((BPW, DB), jnp.float32),
        pltpu.VMEM((BPW, DB), jnp.float32),
        pltpu.VMEM((BPW, D), jnp.float32),
        pltpu.VMEM((BPW, D), jnp.float32),
        pltpu.VMEM((BPW,), jnp.float32),
        pltpu.SemaphoreType.DMA,
        pltpu.SemaphoreType.DMA,
        pltpu.SemaphoreType.DMA,
        pltpu.SemaphoreType.DMA,
        pltpu.SemaphoreType.DMA,
        pltpu.SemaphoreType.DMA,
        pltpu.SemaphoreType.DMA,
    ],
)


def kernel(numeric, cat_0, cat_1, cat_2, cat_3, bkt_0, bkt_1, hist_items,
           W_num, b_num, E_cat_0, E_cat_1, E_cat_2, E_cat_3,
           E_bkt_0, E_bkt_1, E_hist):
    numeric = jnp.pad(numeric, ((0, 0), (0, 3)))
    hist_T = jnp.transpose(hist_items).reshape(HL, B // CH, CH)
    c0 = cat_0.reshape(B // CH, CH)
    c1 = cat_1.reshape(B // CH, CH)
    c2 = cat_2.reshape(B // CH, CH)
    c3 = cat_3.reshape(B // CH, CH)
    b0 = bkt_0.reshape(B // CH, CH)
    b1 = bkt_1.reshape(B // CH, CH)
    w_t = jnp.transpose(W_num)
    outs = _encode(numeric, c0, c1, c2, c3, b0, b1, hist_T, w_t, b_num,
                   E_cat_0, E_cat_1, E_cat_2, E_cat_3, E_bkt_0, E_bkt_1,
                   E_hist)
    return jnp.concatenate(outs, axis=-1)
